# baseline (speedup 1.0000x reference)
import functools

import jax
import jax.numpy as jnp
from jax import lax
from jax.experimental import pallas as pl
from jax.experimental.pallas import tpu as pltpu
from jax.experimental.pallas import tpu_sc as plsc

B = 128
CAND = 5
BW = 5
V = 100000
L = 16
ROWS = B * CAND
NW = 32
RPW = ROWS // NW
BPW = B // NW
U = 5
NIT = V // (16 * U)
OC = 32

_LN2 = 0.6931471805599453
_SQRT2 = 1.4142135623730951
_NEG = -3.0e38
_BIGI = 2 ** 30


def _ln16(s):
    bits = plsc.bitcast(s, jnp.int32)
    e = ((bits >> 23) & 0xFF) - 127
    m = plsc.bitcast((bits & 0x007FFFFF) | 0x3F800000, jnp.float32)
    big = m > _SQRT2
    m = jnp.where(big, m * 0.5, m)
    e = jnp.where(big, e + 1, e)
    z = (m - 1.0) / (m + 1.0)
    z2 = z * z
    p = z * (2.0 + z2 * (2.0 / 3.0 + z2 * (2.0 / 5.0 + z2 * (2.0 / 7.0))))
    return e.astype(jnp.float32) * _LN2 + p


def _merge16(bv, bi, v, vi):
    sv, si = plsc.sort_key_val(v, vi, descending=True)
    rv = lax.rev(sv, (0,))
    ri = lax.rev(si, (0,))
    take = (bv > rv) | ((bv == rv) & (bi <= ri))
    mv = jnp.where(take, bv, rv)
    mi = jnp.where(take, bi, ri)
    return plsc.sort_key_val(mv, mi, descending=True)


_mesh = plsc.VectorSubcoreMesh(core_axis_name="c", subcore_axis_name="s")


def _build(interpret=False):
  @functools.partial(
      pl.kernel,
      out_type=(
          jax.ShapeDtypeStruct((ROWS * OC,), jnp.int32),
          jax.ShapeDtypeStruct((B * 16,), jnp.float32),
      ),
      mesh=_mesh,
      compiler_params=pltpu.CompilerParams(needs_layout_passes=False),
      interpret=interpret,
      scratch_types=[
          pltpu.VMEM((V,), jnp.float32),
          pltpu.VMEM((RPW, 16), jnp.float32),
          pltpu.VMEM((RPW, 16), jnp.int32),
          pltpu.VMEM((RPW * L,), jnp.int32),
          pltpu.VMEM((ROWS,), jnp.float32),
          pltpu.VMEM((RPW * OC,), jnp.int32),
          pltpu.VMEM((BPW * 16,), jnp.float32),
      ],
  )
  def _beam_kernel(scores, lpflat, yflat, out_y, out_nlp,
                   rowbuf, tvals, tidx, yloc, lpv, obuf, nlpbuf):
    wid = lax.axis_index("s") * 2 + lax.axis_index("c")
    base_row = wid * RPW
    pltpu.sync_copy(lpflat, lpv)
    pltpu.sync_copy(yflat.at[pl.ds(base_row * L, RPW * L)], yloc)

    iota = lax.iota(jnp.int32, 16)
    neg = jnp.full((16,), _NEG, jnp.float32)

    def row_body(r, _):
        row = base_row + r
        pltpu.sync_copy(scores.at[pl.ds(row * V, V)], rowbuf)

        def p1(i, carry):
            rm, bv, bi, th = carry
            b0 = i * (16 * U)
            vs = [rowbuf[pl.ds(b0 + 16 * k, 16)] for k in range(U)]
            cm = vs[0]
            for k in range(1, U):
                cm = jnp.maximum(cm, vs[k])
            rm = jnp.maximum(rm, cm)
            pred = jnp.any(cm > th)

            def t_fn(ops):
                bv, bi, b0 = ops[0], ops[1], ops[2]
                for k in range(U):
                    vi = b0 + k * 16 + iota
                    bv, bi = _merge16(bv, bi, ops[3 + k], vi)
                th2 = jnp.broadcast_to(jnp.min(bv), (16,))
                return bv, bi, th2

            bv, bi, th = lax.cond(
                pred, t_fn, lambda ops: (ops[0], ops[1], carry[3]),
                (bv, bi, b0) + tuple(vs))
            return rm, bv, bi, th

        rm0 = neg
        bv0 = neg
        bi0 = jnp.zeros((16,), jnp.int32)
        rm, bv, bi, _ = lax.fori_loop(
            0, NIT, p1, (rm0, bv0, bi0, neg))

        m = jnp.max(rm)
        ms = jnp.broadcast_to(m, (16,))

        def p2(i, s):
            b0 = i * (16 * U)
            for k in range(U):
                s = s + jnp.exp(rowbuf[pl.ds(b0 + 16 * k, 16)] - ms)
            return s

        s = lax.fori_loop(0, NIT, p2, jnp.zeros((16,), jnp.float32))
        ss = jnp.broadcast_to(jnp.sum(s), (16,))
        lse16 = ms + _ln16(ss)
        ov = neg
        oi = jnp.zeros((16,), jnp.int32)
        bvw = bv
        for t in range(BW):
            mt = jnp.max(bvw)
            it = jnp.min(jnp.where(bvw == mt, bi, _BIGI))
            bvw = jnp.where((bvw == mt) & (bi == it), _NEG, bvw)
            ov = jnp.where(iota == t, mt, ov)
            oi = jnp.where(iota == t, it, oi)
        tvals[r, :] = ov - lse16
        tidx[r, :] = oi
        return 0

    lax.fori_loop(0, RPW, row_body, 0)

    for bl in range(BPW):
        b = wid * BPW + bl
        la = iota
        ca = la // 5
        ja = la - ca * 5
        va = (plsc.load_gather(lpv, [b * 5 + ca])
              + plsc.load_gather(tvals, [bl * 5 + ca, ja]))
        lb = iota + 16
        cb = jnp.minimum(lb // 5, 4)
        jb = lb - (lb // 5) * 5
        vb = jnp.where(
            lb < 25,
            plsc.load_gather(lpv, [b * 5 + cb])
            + plsc.load_gather(tvals, [bl * 5 + cb, jb]),
            neg)
        nlp_out = neg
        for t in range(BW):
            mt = jnp.maximum(jnp.max(va), jnp.max(vb))
            it = jnp.minimum(
                jnp.min(jnp.where(va == mt, la, _BIGI)),
                jnp.min(jnp.where(vb == mt, lb, _BIGI)))
            va = jnp.where((va == mt) & (la == it), _NEG, va)
            vb = jnp.where((vb == mt) & (lb == it), _NEG, vb)
            nlp_out = jnp.where(iota == t, mt, nlp_out)
            cc = it // 5
            cjj = it - cc * 5
            lrow = bl * 5 + cc
            obuf[pl.ds((bl * 5 + t) * OC, 16)] = plsc.load_gather(
                yloc, [lrow * L + iota])
            obuf[pl.ds((bl * 5 + t) * OC + 16, 16)] = plsc.load_gather(
                tidx, [jnp.broadcast_to(lrow, (16,)),
                       jnp.broadcast_to(cjj, (16,))])
        nlpbuf[pl.ds(bl * 16, 16)] = nlp_out

    pltpu.sync_copy(obuf, out_y.at[pl.ds(base_row * OC, RPW * OC)])
    pltpu.sync_copy(nlpbuf, out_nlp.at[pl.ds(wid * BPW * 16, BPW * 16)])

  return _beam_kernel


_beam_kernel = _build()


def kernel(next_scores, log_probabilities, Y, beam_width, candidates):
    out_y, out_nlp = _beam_kernel(
        next_scores.reshape(-1), log_probabilities.reshape(-1), Y.reshape(-1))
    return (out_y.reshape(ROWS, OC)[:, :L + 1],
            out_nlp.reshape(B, 16)[:, :BW])
